# baseline (speedup 1.0000x reference)
import math

import jax
import jax.numpy as jnp
from jax.experimental import pallas as pl
from jax.experimental.pallas import tpu as pltpu

N_CELLS = 16384
N_SPOTS = 4096
N_LABELS = 512
BC = 1024
CHUNK = 512
LOG2E = math.log2(math.e)


def _prep(seg_ref, ex_ref, ey_ref, d_ref, onehot_ref, row_ref, bias_ref):
    labs = jax.lax.broadcasted_iota(jnp.int32, (N_SPOTS, N_LABELS), 1)
    onehot_ref[...] = (seg_ref[...] == labs).astype(jnp.bfloat16)

    d = d_ref[0, 0]
    c1 = -LOG2E / (2.0 * d)
    c2 = -jnp.log2(2.0 * jnp.pi * d)
    ex = ex_ref[...]
    ey = ey_ref[...]
    row_ref[...] = c1 * (ex * ex + ey * ey) + c2

    ones_row = jnp.ones((8, N_SPOTS), jnp.bfloat16)
    counts = jnp.dot(ones_row, onehot_ref[...],
                     preferred_element_type=jnp.float32)
    bias_ref[...] = 1e-12 * counts[0:1, :]


def _main(onehot_ref, ax_ref, ay_ref, col_ref, ex_ref, ey_ref, row_ref,
          bias_ref, out_ref):
    ax = ax_ref[...]
    ay = ay_ref[...]
    col = col_ref[...]
    acc = None
    for c in range(N_SPOTS // CHUNK):
        sl = pl.ds(c * CHUNK, CHUNK)
        t = (ax * ex_ref[:, sl] + ay * ey_ref[:, sl]) \
            + (row_ref[:, sl] + col)
        k = jnp.exp2(t).astype(jnp.bfloat16)
        p = jnp.dot(k, onehot_ref[sl, :],
                    preferred_element_type=jnp.float32)
        acc = p if acc is None else acc + p
    out_ref[...] = acc + bias_ref[...]


@jax.jit
def kernel(z, diffusion_constant, encoding_x, encoding_y, segment_ids):
    d = diffusion_constant.astype(jnp.float32)
    c1 = -LOG2E / (2.0 * d)

    zx = z[:, 0:1]
    zy = z[:, 1:2]
    ax = (-2.0 * c1) * zx
    ay = (-2.0 * c1) * zy
    col = c1 * (zx * zx + zy * zy)

    ex = encoding_x.reshape(1, N_SPOTS)
    ey = encoding_y.reshape(1, N_SPOTS)
    seg = segment_ids.reshape(N_SPOTS, 1)

    onehot, row, bias = pl.pallas_call(
        _prep,
        out_shape=(
            jax.ShapeDtypeStruct((N_SPOTS, N_LABELS), jnp.bfloat16),
            jax.ShapeDtypeStruct((1, N_SPOTS), jnp.float32),
            jax.ShapeDtypeStruct((1, N_LABELS), jnp.float32),
        ),
    )(seg, ex, ey, d.reshape(1, 1))

    grid = (N_CELLS // BC,)
    return pl.pallas_call(
        _main,
        grid=grid,
        in_specs=[
            pl.BlockSpec((N_SPOTS, N_LABELS), lambda i: (0, 0)),
            pl.BlockSpec((BC, 1), lambda i: (i, 0)),
            pl.BlockSpec((BC, 1), lambda i: (i, 0)),
            pl.BlockSpec((BC, 1), lambda i: (i, 0)),
            pl.BlockSpec((1, N_SPOTS), lambda i: (0, 0)),
            pl.BlockSpec((1, N_SPOTS), lambda i: (0, 0)),
            pl.BlockSpec((1, N_SPOTS), lambda i: (0, 0)),
            pl.BlockSpec((1, N_LABELS), lambda i: (0, 0)),
        ],
        out_specs=pl.BlockSpec((BC, N_LABELS), lambda i: (i, 0)),
        out_shape=jax.ShapeDtypeStruct((N_CELLS, N_LABELS), jnp.float32),
        compiler_params=pltpu.CompilerParams(
            dimension_semantics=("arbitrary",),
        ),
    )(onehot, ax, ay, col, ex, ey, row, bias)


# speedup vs baseline: 1.1586x; 1.1586x over previous
import math

import jax
import jax.numpy as jnp
from jax.experimental import pallas as pl
from jax.experimental.pallas import tpu as pltpu

N_CELLS = 16384
N_SPOTS = 4096
N_LABELS = 512
BC = 1024
CHUNK = 512
LOG2E = math.log2(math.e)


def _prep(seg_ref, ex_ref, ey_ref, z_ref, d_ref,
          onehot_ref, row_ref, bias_ref, ax_ref, ay_ref, col_ref):
    labs = jax.lax.broadcasted_iota(jnp.int32, (N_SPOTS, N_LABELS), 1)
    onehot_ref[...] = (seg_ref[...] == labs).astype(jnp.bfloat16)

    d = d_ref[0, 0]
    c1 = -LOG2E / (2.0 * d)
    c2 = -jnp.log2(2.0 * jnp.pi * d)
    ex = ex_ref[...]
    ey = ey_ref[...]
    row_ref[...] = c1 * (ex * ex + ey * ey) + c2

    zx = z_ref[:, 0:1]
    zy = z_ref[:, 1:2]
    ax_ref[...] = (-2.0 * c1) * zx
    ay_ref[...] = (-2.0 * c1) * zy
    col_ref[...] = c1 * (zx * zx + zy * zy)

    ones_row = jnp.ones((8, N_SPOTS), jnp.bfloat16)
    counts = jnp.dot(ones_row, onehot_ref[...],
                     preferred_element_type=jnp.float32)
    bias_ref[...] = 1e-12 * counts[0:1, :]


def _main(onehot_ref, ax_ref, ay_ref, col_ref, ex_ref, ey_ref, row_ref,
          bias_ref, out_ref):
    ax = ax_ref[...]
    ay = ay_ref[...]
    col = col_ref[...]

    parts = []
    for c in range(N_SPOTS // CHUNK):
        sl = pl.ds(c * CHUNK, CHUNK)
        t = (ax * ex_ref[:, sl] + ay * ey_ref[:, sl]) \
            + (row_ref[:, sl] + col)
        k = jnp.exp2(t).astype(jnp.bfloat16)
        parts.append(jnp.dot(k, onehot_ref[sl, :],
                             preferred_element_type=jnp.float32))
    while len(parts) > 1:
        parts = [a + b for a, b in zip(parts[::2], parts[1::2])]
    out_ref[...] = parts[0] + bias_ref[...]


@jax.jit
def kernel(z, diffusion_constant, encoding_x, encoding_y, segment_ids):
    ex = encoding_x.reshape(1, N_SPOTS)
    ey = encoding_y.reshape(1, N_SPOTS)
    seg = segment_ids.reshape(N_SPOTS, 1)
    d2d = diffusion_constant.reshape(1, 1).astype(jnp.float32)

    onehot, row, bias, ax, ay, col = pl.pallas_call(
        _prep,
        out_shape=(
            jax.ShapeDtypeStruct((N_SPOTS, N_LABELS), jnp.bfloat16),
            jax.ShapeDtypeStruct((1, N_SPOTS), jnp.float32),
            jax.ShapeDtypeStruct((1, N_LABELS), jnp.float32),
            jax.ShapeDtypeStruct((N_CELLS, 1), jnp.float32),
            jax.ShapeDtypeStruct((N_CELLS, 1), jnp.float32),
            jax.ShapeDtypeStruct((N_CELLS, 1), jnp.float32),
        ),
    )(seg, ex, ey, z, d2d)

    grid = (N_CELLS // BC,)
    return pl.pallas_call(
        _main,
        grid=grid,
        in_specs=[
            pl.BlockSpec((N_SPOTS, N_LABELS), lambda i: (0, 0)),
            pl.BlockSpec((BC, 1), lambda i: (i, 0)),
            pl.BlockSpec((BC, 1), lambda i: (i, 0)),
            pl.BlockSpec((BC, 1), lambda i: (i, 0)),
            pl.BlockSpec((1, N_SPOTS), lambda i: (0, 0)),
            pl.BlockSpec((1, N_SPOTS), lambda i: (0, 0)),
            pl.BlockSpec((1, N_SPOTS), lambda i: (0, 0)),
            pl.BlockSpec((1, N_LABELS), lambda i: (0, 0)),
        ],
        out_specs=pl.BlockSpec((BC, N_LABELS), lambda i: (i, 0)),
        out_shape=jax.ShapeDtypeStruct((N_CELLS, N_LABELS), jnp.float32),
        compiler_params=pltpu.CompilerParams(
            dimension_semantics=("arbitrary",),
        ),
    )(onehot, ax, ay, col, ex, ey, row, bias)
